# baseline (device time: 51688 ns/iter reference)
import jax
import jax.numpy as jnp
from jax import lax
from jax.experimental import pallas as pl
from jax.experimental.pallas import tpu as pltpu

N_DEV = 4
N_LAYERS = 3
N_PEER = N_DEV - 1


def kernel(x, Win0, Wout0, Win1, Wout1, Win2, Wout2):
    m_per, d = x.shape
    m = N_DEV * m_per

    def body(x_ref, win0_hbm, wout0_hbm, win1_hbm, wout1_hbm, win2_hbm,
             wout2_hbm, out_ref,
             xbuf, rs, rs_src, obuf, win_v, wout_v,
             ag_send, ag_recv, rs_send, rs_recv, out_send, out_recv, w_sem):
        my = lax.axis_index("i")
        pending_sends = []

        w_hbm = ((win0_hbm, wout0_hbm), (win1_hbm, wout1_hbm),
                 (win2_hbm, wout2_hbm))
        w_dmas = []
        for l in range(N_LAYERS):
            din = pltpu.make_async_copy(w_hbm[l][0], win_v.at[l],
                                        w_sem.at[2 * l])
            dout = pltpu.make_async_copy(w_hbm[l][1], wout_v.at[l],
                                         w_sem.at[2 * l + 1])
            din.start()
            dout.start()
            w_dmas.append((din, dout))

        def chunk_copy(src_ref, dst_ref, send_sem, recv_sem, peer):
            return pltpu.make_async_remote_copy(
                src_ref=src_ref, dst_ref=dst_ref,
                send_sem=send_sem, recv_sem=recv_sem,
                device_id=(peer,), device_id_type=pl.DeviceIdType.MESH,
            )

        def wait_chunk_recv(dst_ref, recv_sem):
            chunk_copy(x_ref, dst_ref, ag_send.at[0], recv_sem, my).wait_recv()

        barrier_sem = pltpu.get_barrier_semaphore()
        for k in range(1, N_DEV):
            pl.semaphore_signal(
                barrier_sem, inc=1,
                device_id=(lax.rem(my + k, N_DEV),),
                device_id_type=pl.DeviceIdType.MESH,
            )
        pl.semaphore_wait(barrier_sem, N_DEV - 1)

        xbuf[0, N_PEER] = x_ref[...]
        for k in range(1, N_DEV):
            peer = lax.rem(my + k, N_DEV)
            s = N_PEER - k
            rdma = chunk_copy(x_ref, xbuf.at[0, s],
                              ag_send.at[s], ag_recv.at[s], peer)
            rdma.start()
            pending_sends.append(rdma)

        def layer_partial(win, wout, xc):
            h = jnp.maximum(
                jnp.dot(xc, win[...], preferred_element_type=jnp.float32), 0.0)
            return jnp.dot(h, wout[...], preferred_element_type=jnp.float32)

        for l in range(N_LAYERS):
            w_dmas[l][0].wait()
            w_dmas[l][1].wait()
            win, wout = win_v.at[l], wout_v.at[l]
            par = l % 2
            for s in range(N_PEER):
                wait_chunk_recv(xbuf.at[par, s], ag_recv.at[l * 3 + s])
                partial = layer_partial(win, wout, xbuf[par, s])
                rs_src[l * 3 + s] = partial
                owner = lax.rem(my + 1 + s, N_DEV)
                rdma = chunk_copy(rs_src.at[l * 3 + s], rs.at[s],
                                  rs_send.at[l * 3 + s],
                                  rs_recv.at[l * 3 + s], owner)
                rdma.start()
                pending_sends.append(rdma)
            total = layer_partial(win, wout, xbuf[par, N_PEER])
            for s in range(N_PEER):
                wait_chunk_recv(rs.at[s], rs_recv.at[l * 3 + s])
            total = total + rs[0] + rs[1] + rs[2]

            if l < N_LAYERS - 1:
                nxt = 1 - par
                xbuf[nxt, N_PEER] = total
                for k in range(1, N_DEV):
                    peer = lax.rem(my + k, N_DEV)
                    s = N_PEER - k
                    rdma = chunk_copy(xbuf.at[nxt, N_PEER], xbuf.at[nxt, s],
                                      ag_send.at[(l + 1) * 3 + s],
                                      ag_recv.at[(l + 1) * 3 + s], peer)
                    rdma.start()
                    pending_sends.append(rdma)
            else:
                obuf[0] = total
                for k in range(1, N_DEV):
                    peer = lax.rem(my + k, N_DEV)
                    rdma = chunk_copy(obuf.at[0], obuf.at[N_DEV - k],
                                      out_send.at[k - 1],
                                      out_recv.at[k - 1], peer)
                    rdma.start()
                    pending_sends.append(rdma)
                out_ref[pl.ds(my * m_per, m_per), :] = total
                for r in range(1, N_DEV):
                    wait_chunk_recv(obuf.at[r], out_recv.at[N_PEER - r])
                    origin = lax.rem(my + r, N_DEV)
                    out_ref[pl.ds(origin * m_per, m_per), :] = obuf[r]

        for rdma in pending_sends:
            rdma.wait_send()

    return pl.pallas_call(
        body,
        out_shape=jax.ShapeDtypeStruct((m, d), jnp.float32),
        in_specs=[pl.BlockSpec(memory_space=pltpu.VMEM)]
        + [pl.BlockSpec(memory_space=pl.ANY)] * 6,
        out_specs=pl.BlockSpec(memory_space=pltpu.VMEM),
        scratch_shapes=[
            pltpu.VMEM((2, N_DEV, m_per, d), jnp.float32),
            pltpu.VMEM((N_PEER, m_per, d), jnp.float32),
            pltpu.VMEM((N_LAYERS * N_PEER, m_per, d), jnp.float32),
            pltpu.VMEM((N_DEV, m_per, d), jnp.float32),
            pltpu.VMEM((N_LAYERS,) + Win0.shape, jnp.float32),
            pltpu.VMEM((N_LAYERS,) + Wout0.shape, jnp.float32),
            pltpu.SemaphoreType.DMA((N_LAYERS * N_PEER,)),
            pltpu.SemaphoreType.DMA((N_LAYERS * N_PEER,)),
            pltpu.SemaphoreType.DMA((N_LAYERS * N_PEER,)),
            pltpu.SemaphoreType.DMA((N_LAYERS * N_PEER,)),
            pltpu.SemaphoreType.DMA((N_PEER,)),
            pltpu.SemaphoreType.DMA((N_PEER,)),
            pltpu.SemaphoreType.DMA((2 * N_LAYERS,)),
        ],
        compiler_params=pltpu.CompilerParams(collective_id=0),
    )(x, Win0, Wout0, Win1, Wout1, Win2, Wout2)


# device time: 38624 ns/iter; 1.3382x vs baseline; 1.3382x over previous
import jax
import jax.numpy as jnp
from jax import lax
from jax.experimental import pallas as pl
from jax.experimental.pallas import tpu as pltpu

N_DEV = 4
N_LAYERS = 3
N_PEER = N_DEV - 1


def kernel(x, Win0, Wout0, Win1, Wout1, Win2, Wout2):
    m_per, d = x.shape
    m = N_DEV * m_per

    def body(x_ref, win0, wout0, win1, wout1, win2, wout2, out_ref,
             xbuf, xb, rs, rs_src, obuf,
             ag_send, ag_recv, rs_send, rs_recv, out_send, out_recv):
        my = lax.axis_index("i")
        pending_sends = []

        def chunk_copy(src_ref, dst_ref, send_sem, recv_sem, peer):
            return pltpu.make_async_remote_copy(
                src_ref=src_ref, dst_ref=dst_ref,
                send_sem=send_sem, recv_sem=recv_sem,
                device_id=(peer,), device_id_type=pl.DeviceIdType.MESH,
            )

        def wait_chunk_recv(dst_ref, recv_sem):
            chunk_copy(xb, dst_ref, ag_send.at[0], recv_sem, my).wait_recv()

        barrier_sem = pltpu.get_barrier_semaphore()
        for k in range(1, N_DEV):
            pl.semaphore_signal(
                barrier_sem, inc=1,
                device_id=(lax.rem(my + k, N_DEV),),
                device_id_type=pl.DeviceIdType.MESH,
            )
        pl.semaphore_wait(barrier_sem, N_DEV - 1)

        xb[...] = x_ref[...].astype(jnp.bfloat16)
        for k in (2, 1, 3):
            peer = lax.rem(my + k, N_DEV)
            s = N_PEER - k
            rdma = chunk_copy(xb, xbuf.at[0, s],
                              ag_send.at[s], ag_recv.at[s], peer)
            rdma.start()
            pending_sends.append(rdma)

        weights = ((win0, wout0), (win1, wout1), (win2, wout2))

        def layer_partial(l, xc):
            win, wout = weights[l]
            h = jnp.maximum(
                jnp.dot(xc, win[...], preferred_element_type=jnp.float32), 0.0)
            return jnp.dot(h.astype(jnp.bfloat16), wout[...],
                           preferred_element_type=jnp.float32)

        own_x = x_ref[...]
        for l in range(N_LAYERS):
            par = l % 2
            win, wout = weights[l]
            h_own = jnp.maximum(
                jnp.dot(own_x.astype(jnp.bfloat16), win[...],
                        preferred_element_type=jnp.float32),
                0.0)
            for s in range(N_PEER):
                wait_chunk_recv(xbuf.at[par, s], ag_recv.at[l * 3 + s])
            xr = xbuf[par, 0:N_PEER].reshape(N_PEER * m_per, d)
            p_remote = layer_partial(l, xr)
            for s in (1, 0, 2):
                rs_src[l * 3 + s] = (
                    p_remote[s * m_per:(s + 1) * m_per].astype(jnp.bfloat16))
                owner = lax.rem(my + 1 + s, N_DEV)
                rdma = chunk_copy(rs_src.at[l * 3 + s], rs.at[s],
                                  rs_send.at[l * 3 + s],
                                  rs_recv.at[l * 3 + s], owner)
                rdma.start()
                pending_sends.append(rdma)
            own_partial = jnp.dot(h_own.astype(jnp.bfloat16), wout[...],
                                  preferred_element_type=jnp.float32)
            for s in range(N_PEER):
                wait_chunk_recv(rs.at[s], rs_recv.at[l * 3 + s])
            total = (own_partial
                     + rs[0].astype(jnp.float32)
                     + rs[1].astype(jnp.float32)
                     + rs[2].astype(jnp.float32))

            if l < N_LAYERS - 1:
                nxt = 1 - par
                xbuf[nxt, N_PEER] = total.astype(jnp.bfloat16)
                for k in (2, 1, 3):
                    peer = lax.rem(my + k, N_DEV)
                    s = N_PEER - k
                    rdma = chunk_copy(xbuf.at[nxt, N_PEER], xbuf.at[nxt, s],
                                      ag_send.at[(l + 1) * 3 + s],
                                      ag_recv.at[(l + 1) * 3 + s], peer)
                    rdma.start()
                    pending_sends.append(rdma)
                own_x = total
            else:
                obuf[0] = total.astype(jnp.bfloat16)
                for k in (2, 1, 3):
                    peer = lax.rem(my + k, N_DEV)
                    rdma = chunk_copy(obuf.at[0], obuf.at[N_DEV - k],
                                      out_send.at[k - 1],
                                      out_recv.at[k - 1], peer)
                    rdma.start()
                    pending_sends.append(rdma)
                out_ref[pl.ds(my * m_per, m_per), :] = total
                for r in range(1, N_DEV):
                    wait_chunk_recv(obuf.at[r], out_recv.at[N_PEER - r])
                    origin = lax.rem(my + r, N_DEV)
                    out_ref[pl.ds(origin * m_per, m_per), :] = (
                        obuf[r].astype(jnp.float32))

        for rdma in pending_sends:
            rdma.wait_send()

    return pl.pallas_call(
        body,
        out_shape=jax.ShapeDtypeStruct((m, d), jnp.float32),
        in_specs=[pl.BlockSpec(memory_space=pltpu.VMEM)] * 7,
        out_specs=pl.BlockSpec(memory_space=pltpu.VMEM),
        scratch_shapes=[
            pltpu.VMEM((2, N_DEV, m_per, d), jnp.bfloat16),
            pltpu.VMEM((m_per, d), jnp.bfloat16),
            pltpu.VMEM((N_PEER, m_per, d), jnp.bfloat16),
            pltpu.VMEM((N_LAYERS * N_PEER, m_per, d), jnp.bfloat16),
            pltpu.VMEM((N_DEV, m_per, d), jnp.bfloat16),
            pltpu.SemaphoreType.DMA((N_LAYERS * N_PEER,)),
            pltpu.SemaphoreType.DMA((N_LAYERS * N_PEER,)),
            pltpu.SemaphoreType.DMA((N_LAYERS * N_PEER,)),
            pltpu.SemaphoreType.DMA((N_LAYERS * N_PEER,)),
            pltpu.SemaphoreType.DMA((N_PEER,)),
            pltpu.SemaphoreType.DMA((N_PEER,)),
        ],
        compiler_params=pltpu.CompilerParams(collective_id=0),
    )(x, *(w.astype(jnp.bfloat16)
           for w in (Win0, Wout0, Win1, Wout1, Win2, Wout2)))
